# baseline (speedup 1.0000x reference)
import functools

import jax
import jax.numpy as jnp
from jax import lax
from jax.experimental import pallas as pl
from jax.experimental.pallas import tpu as pltpu
from jax.experimental.pallas import tpu_sc as plsc

VOCAB = 100000
EMBED_DIM = 32
BATCH = 1024


@functools.cache
def _make_sc_gather(B):
    info = plsc.get_sparse_core_info()
    nc, ns, L = info.num_cores, info.num_subcores, info.num_lanes
    nw = nc * ns
    assert nw == EMBED_DIM
    mesh = plsc.VectorSubcoreMesh(core_axis_name="c", subcore_axis_name="s")

    @functools.partial(
        pl.kernel,
        mesh=mesh,
        out_type=jax.ShapeDtypeStruct((EMBED_DIM, B), jnp.float32),
        scratch_types=[
            pltpu.VMEM((VOCAB,), jnp.float32),
            pltpu.VMEM((B,), jnp.int32),
            pltpu.VMEM((B,), jnp.float32),
        ],
        compiler_params=pltpu.CompilerParams(needs_layout_passes=False),
    )
    def gather_kernel(idx_hbm, tablet_hbm, out_hbm, row_v, idx_v, xrow_v):
        d = lax.axis_index("s") * nc + lax.axis_index("c")
        pltpu.sync_copy(idx_hbm, idx_v)
        pltpu.sync_copy(tablet_hbm.at[d], row_v)
        for c in range(B // L):
            sl = pl.ds(c * L, L)
            xrow_v[sl] = plsc.load_gather(row_v, [idx_v[sl]])
        pltpu.sync_copy(xrow_v, out_hbm.at[d])

    return gather_kernel


_BV = 1024
_VBLK = pl.cdiv(VOCAB, _BV)
_TAIL = VOCAB - (_VBLK - 1) * _BV
_NBUF = 5


def _proj_body(wt_ref, xt_ref, b_ref, out_hbm, scratch, sems):
    i = pl.program_id(0)
    buf = lax.rem(i, _NBUF)

    @pl.when(i >= _NBUF)
    def _wait_buffer_free():
        pltpu.make_async_copy(
            scratch.at[buf],
            out_hbm.at[pl.ds((i - _NBUF) * _BV, _BV), :],
            sems.at[buf],
        ).wait()

    acc = lax.dot_general(
        wt_ref[...], xt_ref[...],
        dimension_numbers=(((0,), (0,)), ((), ())),
        preferred_element_type=jnp.float32,
    )
    bias = lax.broadcast_in_dim(b_ref[...], (_BV, BATCH), (0,))
    scratch[buf] = acc + bias

    @pl.when(i < _VBLK - 1)
    def _start_full():
        pltpu.make_async_copy(
            scratch.at[buf],
            out_hbm.at[pl.ds(i * _BV, _BV), :],
            sems.at[buf],
        ).start()

    @pl.when(i == _VBLK - 1)
    def _start_tail_and_drain():
        pltpu.make_async_copy(
            scratch.at[buf, pl.ds(0, _TAIL), :],
            out_hbm.at[pl.ds((_VBLK - 1) * _BV, _TAIL), :],
            sems.at[buf],
        ).start()
        for s in range(_VBLK - _NBUF, _VBLK):
            bs = s % _NBUF
            if s == _VBLK - 1:
                pltpu.make_async_copy(
                    scratch.at[bs, pl.ds(0, _TAIL), :],
                    out_hbm.at[pl.ds(s * _BV, _TAIL), :],
                    sems.at[bs],
                ).wait()
            else:
                pltpu.make_async_copy(
                    scratch.at[bs],
                    out_hbm.at[pl.ds(s * _BV, _BV), :],
                    sems.at[bs],
                ).wait()


def _projection(xt, W, b):
    out_t = pl.pallas_call(
        _proj_body,
        grid=(_VBLK,),
        in_specs=[
            pl.BlockSpec((EMBED_DIM, _BV), lambda i: (0, i)),
            pl.BlockSpec((EMBED_DIM, BATCH), lambda i: (0, 0)),
            pl.BlockSpec((_BV,), lambda i: (i,)),
        ],
        out_specs=pl.BlockSpec(memory_space=pltpu.MemorySpace.HBM),
        out_shape=jax.ShapeDtypeStruct((VOCAB, BATCH), jnp.float32),
        scratch_shapes=[
            pltpu.VMEM((_NBUF, _BV, BATCH), jnp.float32),
            pltpu.SemaphoreType.DMA((_NBUF,)),
        ],
        compiler_params=pltpu.CompilerParams(
            dimension_semantics=("arbitrary",),
            vmem_limit_bytes=63 * 1024 * 1024,
        ),
    )(W.T, xt, b)
    return out_t.T


def kernel(input_idx, emb_table, W, b):
    idx = input_idx.astype(jnp.int32)
    xt = _make_sc_gather(BATCH)(idx, emb_table.T)
    return _projection(xt, W, b)


# speedup vs baseline: 1.0259x; 1.0259x over previous
import functools

import jax
import jax.numpy as jnp
from jax import lax
from jax.experimental import pallas as pl
from jax.experimental.pallas import tpu as pltpu
from jax.experimental.pallas import tpu_sc as plsc

VOCAB = 100000
EMBED_DIM = 32
BATCH = 1024


@functools.cache
def _make_sc_gather(B):
    info = plsc.get_sparse_core_info()
    nc, ns, L = info.num_cores, info.num_subcores, info.num_lanes
    nw = nc * ns
    assert nw == EMBED_DIM
    mesh = plsc.VectorSubcoreMesh(core_axis_name="c", subcore_axis_name="s")

    @functools.partial(
        pl.kernel,
        mesh=mesh,
        out_type=jax.ShapeDtypeStruct((EMBED_DIM, B), jnp.float32),
        scratch_types=[
            pltpu.VMEM((VOCAB,), jnp.float32),
            pltpu.VMEM((B,), jnp.int32),
            pltpu.VMEM((B,), jnp.float32),
            pltpu.SemaphoreType.DMA,
            pltpu.SemaphoreType.DMA,
        ],
        compiler_params=pltpu.CompilerParams(needs_layout_passes=False),
    )
    def gather_kernel(idx_hbm, tablet_hbm, out_hbm, row_v, idx_v, xrow_v,
                      sem_i, sem_a):
        d = lax.axis_index("s") * nc + lax.axis_index("c")
        ci = pltpu.async_copy(idx_hbm, idx_v, sem_i)
        ca = pltpu.async_copy(tablet_hbm.at[d], row_v, sem_a)
        ci.wait(); ca.wait()
        for c in range(B // L):
            sl = pl.ds(c * L, L)
            xrow_v[sl] = plsc.load_gather(row_v, [idx_v[sl]])
        pltpu.sync_copy(xrow_v, out_hbm.at[d])

    return gather_kernel


_BV = 2048
_VBLK = pl.cdiv(VOCAB, _BV)
_TAIL = VOCAB - (_VBLK - 1) * _BV
_NBUF = 3


def _proj_body(wt_ref, xt_ref, b_ref, out_hbm, scratch, sems):
    i = pl.program_id(0)
    buf = lax.rem(i, _NBUF)

    @pl.when(i >= _NBUF)
    def _wait_buffer_free():
        pltpu.make_async_copy(
            scratch.at[buf],
            out_hbm.at[pl.ds((i - _NBUF) * _BV, _BV), :],
            sems.at[buf],
        ).wait()

    acc = lax.dot_general(
        wt_ref[...], xt_ref[...],
        dimension_numbers=(((0,), (0,)), ((), ())),
        preferred_element_type=jnp.float32,
    )
    bias = lax.broadcast_in_dim(b_ref[...], (_BV, BATCH), (0,))
    scratch[buf] = acc + bias

    @pl.when(i < _VBLK - 1)
    def _start_full():
        pltpu.make_async_copy(
            scratch.at[buf],
            out_hbm.at[pl.ds(i * _BV, _BV), :],
            sems.at[buf],
        ).start()

    @pl.when(i == _VBLK - 1)
    def _start_tail_and_drain():
        pltpu.make_async_copy(
            scratch.at[buf, pl.ds(0, _TAIL), :],
            out_hbm.at[pl.ds((_VBLK - 1) * _BV, _TAIL), :],
            sems.at[buf],
        ).start()
        for s in range(_VBLK - _NBUF, _VBLK):
            bs = s % _NBUF
            if s == _VBLK - 1:
                pltpu.make_async_copy(
                    scratch.at[bs, pl.ds(0, _TAIL), :],
                    out_hbm.at[pl.ds(s * _BV, _TAIL), :],
                    sems.at[bs],
                ).wait()
            else:
                pltpu.make_async_copy(
                    scratch.at[bs],
                    out_hbm.at[pl.ds(s * _BV, _BV), :],
                    sems.at[bs],
                ).wait()


def _projection(xt, W, b):
    out_t = pl.pallas_call(
        _proj_body,
        grid=(_VBLK,),
        in_specs=[
            pl.BlockSpec((EMBED_DIM, _BV), lambda i: (0, i)),
            pl.BlockSpec((EMBED_DIM, BATCH), lambda i: (0, 0)),
            pl.BlockSpec((_BV,), lambda i: (i,)),
        ],
        out_specs=pl.BlockSpec(memory_space=pltpu.MemorySpace.HBM),
        out_shape=jax.ShapeDtypeStruct((VOCAB, BATCH), jnp.float32),
        scratch_shapes=[
            pltpu.VMEM((_NBUF, _BV, BATCH), jnp.float32),
            pltpu.SemaphoreType.DMA((_NBUF,)),
        ],
        compiler_params=pltpu.CompilerParams(
            dimension_semantics=("arbitrary",),
            vmem_limit_bytes=63 * 1024 * 1024,
        ),
    )(W.T, xt, b)
    return out_t.T


def kernel(input_idx, emb_table, W, b):
    idx = input_idx.astype(jnp.int32)
    xt = _make_sc_gather(BATCH)(idx, emb_table.T)
    return _projection(xt, W, b)
